# baseline (speedup 1.0000x reference)
import functools
import jax
import jax.numpy as jnp
from jax import lax
from jax.experimental import pallas as pl
from jax.experimental.pallas import tpu as pltpu
from jax.experimental.pallas import tpu_sc as plsc

_K = 50
_B = 128
_N = 32768
_NV = _N // 16
_NW = 32
_RPW = _B // _NW
_MIN32 = -2147483648


def _sc_body(x_hbm, out_hbm, xv, cb, hist, tot, ss):
    nc = 2
    wid = lax.axis_index("s") * nc + lax.axis_index("c")
    lane = lax.iota(jnp.int32, 16)
    lane_c = lane * 256
    ones = jnp.ones((16,), jnp.int32)
    zeros = jnp.zeros((16,), jnp.int32)
    s24 = jnp.full((16,), 24, jnp.int32)
    s16 = jnp.full((16,), 16, jnp.int32)
    s8 = jnp.full((16,), 8, jnp.int32)
    m255 = jnp.full((16,), 255, jnp.int32)
    neg_inf = jnp.full((16,), -jnp.inf, jnp.float32)

    def zero_hist():
        def zh(i, c):
            base = i * 256
            for u in range(16):
                hist[pl.ds(base + u * 16, 16)] = zeros
            return c
        lax.fori_loop(0, 16, zh, 0)

    def pick(k_rem):
        def tj(j, c):
            base = j * 16
            acc = hist[pl.ds(base, 16)]
            for l in range(1, 16):
                acc = acc + hist[pl.ds(l * 256 + base, 16)]
            tot[pl.ds(base, 16)] = acc
            return c
        lax.fori_loop(0, 16, tj, 0)

        def sj(i, carry):
            c_above, cnt = carry
            j = 15 - i
            v = tot[pl.ds(j * 16, 16)]
            cum = plsc.cumsum(v)
            s = jnp.sum(v)
            ssv = (s + c_above) - cum + v
            ss[pl.ds(j * 16, 16)] = ssv
            cnt = cnt + jnp.sum((ssv >= k_rem).astype(jnp.int32))
            return (c_above + s, cnt)

        _, cnt = lax.fori_loop(0, 16, sj, (jnp.int32(0), jnp.int32(0)))
        dstar = cnt - 1
        t_d = tot[pl.ds(dstar, 16)][0]
        ss_d = ss[pl.ds(dstar, 16)][0]
        k_next = k_rem - (ss_d - t_d)
        return dstar, t_d, k_next

    def do_row(r, carry):
        row = wid * _RPW + r
        pltpu.sync_copy(x_hbm.at[row], xv)

        zero_hist()

        def h0(i, c):
            v = xv[pl.ds(i * 16, 16)]
            bi = lax.bitcast_convert_type(v, jnp.int32)
            key = bi ^ ((bi >> 31) | _MIN32)
            d0 = lax.shift_right_logical(key, s24)
            plsc.addupdate_scatter(hist, [lane_c + d0], ones)
            return c

        lax.fori_loop(0, _NV, h0, 0)
        d0s, n1, k1 = pick(jnp.int32(_K))

        zero_hist()

        def r1(i, base):
            v = xv[pl.ds(i * 16, 16)]
            bi = lax.bitcast_convert_type(v, jnp.int32)
            key = bi ^ ((bi >> 31) | _MIN32)
            d0 = lax.shift_right_logical(key, s24)
            m = d0 == d0s
            plsc.store_compressed(cb.at[pl.ds(base, 16)], key, mask=m)
            d1 = lax.shift_right_logical(key, s16) & m255
            plsc.addupdate_scatter(hist, [lane_c + d1], ones, mask=m)
            return base + jnp.sum(m.astype(jnp.int32))

        lax.fori_loop(0, _NV, r1, jnp.int32(0))
        d1s, n2, k2 = pick(k1)

        zero_hist()
        nv1 = (n1 + 15) // 16

        def r2(i, base):
            kv = cb[pl.ds(i * 16, 16)]
            valid = (i * 16 + lane) < n1
            d1 = lax.shift_right_logical(kv, s16) & m255
            m = (d1 == d1s) & valid
            plsc.store_compressed(cb.at[pl.ds(base, 16)], kv, mask=m)
            d2 = lax.shift_right_logical(kv, s8) & m255
            plsc.addupdate_scatter(hist, [lane_c + d2], ones, mask=m)
            return base + jnp.sum(m.astype(jnp.int32))

        lax.fori_loop(0, nv1, r2, jnp.int32(0))
        d2s, n3, k3 = pick(k2)

        zero_hist()
        nv2 = (n2 + 15) // 16

        def r3(i, c):
            kv = cb[pl.ds(i * 16, 16)]
            valid = (i * 16 + lane) < n2
            d2 = lax.shift_right_logical(kv, s8) & m255
            m = (d2 == d2s) & valid
            d3 = kv & m255
            plsc.addupdate_scatter(hist, [lane_c + d3], ones, mask=m)
            return c

        lax.fori_loop(0, nv2, r3, 0)
        d3s, _, _ = pick(k3)

        tkey = (d0s * 16777216) + (d1s * 65536) + (d2s * 256) + d3s
        tbits = jnp.where(tkey < 0, tkey ^ _MIN32, ~tkey)
        tfv = lax.bitcast_convert_type(jnp.broadcast_to(tbits, (16,)), jnp.float32)

        def w(i, c):
            v = xv[pl.ds(i * 16, 16)]
            xv[pl.ds(i * 16, 16)] = jnp.where(v < tfv, neg_inf, v)
            return c

        lax.fori_loop(0, _NV, w, 0)
        pltpu.sync_copy(xv, out_hbm.at[row])
        return carry

    lax.fori_loop(0, _RPW, do_row, 0)


@jax.jit
def kernel(x):
    mesh = plsc.VectorSubcoreMesh(
        core_axis_name="c", subcore_axis_name="s", num_cores=2, num_subcores=16
    )
    f = pl.kernel(
        _sc_body,
        out_type=jax.ShapeDtypeStruct((_B, _N), jnp.float32),
        mesh=mesh,
        compiler_params=pltpu.CompilerParams(needs_layout_passes=False),
        scratch_types=[
            pltpu.VMEM((_N,), jnp.float32),
            pltpu.VMEM((_N + 16,), jnp.int32),
            pltpu.VMEM((4096,), jnp.int32),
            pltpu.VMEM((272,), jnp.int32),
            pltpu.VMEM((272,), jnp.int32),
        ],
    )
    return f(x)


# speedup vs baseline: 1.1576x; 1.1576x over previous
import functools
import jax
import jax.numpy as jnp
from jax import lax
from jax.experimental import pallas as pl
from jax.experimental.pallas import tpu as pltpu
from jax.experimental.pallas import tpu_sc as plsc

_K = 50
_B = 128
_N = 32768
_NV = _N // 16
_NW = 32
_RPW = _B // _NW
_MIN32 = -2147483648
_U = 8
_UC = 4


def _sc_body(x_hbm, out_hbm, xv, cb, hist, tot, ss):
    nc = 2
    wid = lax.axis_index("s") * nc + lax.axis_index("c")
    lane = lax.iota(jnp.int32, 16)
    lane_c = lane * 256
    ones = jnp.ones((16,), jnp.int32)
    zeros = jnp.zeros((16,), jnp.int32)
    s24 = jnp.full((16,), 24, jnp.int32)
    s16 = jnp.full((16,), 16, jnp.int32)
    s8 = jnp.full((16,), 8, jnp.int32)
    m255 = jnp.full((16,), 255, jnp.int32)
    neg_inf = jnp.full((16,), -jnp.inf, jnp.float32)

    def key_of(v):
        bi = lax.bitcast_convert_type(v, jnp.int32)
        return bi ^ ((bi >> 31) | _MIN32)

    def zero_hist():
        def zh(i, c):
            base = i * 256
            for u in range(16):
                hist[pl.ds(base + u * 16, 16)] = zeros
            return c
        lax.fori_loop(0, 16, zh, 0)

    def pick(k_rem):
        def tj(j, c):
            base = j * 16
            acc = hist[pl.ds(base, 16)]
            hist[pl.ds(base, 16)] = zeros
            for l in range(1, 16):
                off = l * 256 + base
                acc = acc + hist[pl.ds(off, 16)]
                hist[pl.ds(off, 16)] = zeros
            tot[pl.ds(base, 16)] = acc
            return c
        lax.fori_loop(0, 16, tj, 0)

        def sj(i, carry):
            c_above, cnt_v = carry
            j = 15 - i
            v = tot[pl.ds(j * 16, 16)]
            cum = plsc.cumsum(v)
            s = cum[15]
            ssv = (s + c_above) - cum + v
            ss[pl.ds(j * 16, 16)] = ssv
            cnt_v = cnt_v + plsc.all_reduce_population_count(ssv >= k_rem)
            return (c_above + s, cnt_v)

        _, cnt_v = lax.fori_loop(0, 16, sj, (jnp.int32(0), zeros))
        dstar = cnt_v[0] - 1
        t_d = tot[pl.ds(dstar, 16)][0]
        ss_d = ss[pl.ds(dstar, 16)][0]
        k_next = k_rem - (ss_d - t_d)
        return dstar, t_d, k_next

    def do_row(r, carry):
        row = wid * _RPW + r
        pltpu.sync_copy(x_hbm.at[row], xv)

        def h0(g, c):
            for u in range(_U):
                v = xv[pl.ds((g * _U + u) * 16, 16)]
                d0 = lax.shift_right_logical(key_of(v), s24)
                plsc.addupdate_scatter(hist, [lane_c + d0], ones)
            return c

        lax.fori_loop(0, _NV // _U, h0, 0)
        d0s, n1, k1 = pick(jnp.int32(_K))

        def r1(g, base_v):
            for u in range(_U):
                v = xv[pl.ds((g * _U + u) * 16, 16)]
                key = key_of(v)
                d0 = lax.shift_right_logical(key, s24)
                m = d0 == d0s
                plsc.store_compressed(cb.at[pl.ds(base_v[0], 16)], key, mask=m)
                d1 = lax.shift_right_logical(key, s16) & m255
                plsc.addupdate_scatter(hist, [lane_c + d1], ones, mask=m)
                base_v = base_v + plsc.all_reduce_population_count(m)
            return base_v

        lax.fori_loop(0, _NV // _U, r1, zeros)
        d1s, n2, k2 = pick(k1)

        def r2(g, base_v):
            for u in range(_UC):
                i = g * _UC + u
                kv = cb[pl.ds(i * 16, 16)]
                valid = (i * 16 + lane) < n1
                d1 = lax.shift_right_logical(kv, s16) & m255
                m = (d1 == d1s) & valid
                plsc.store_compressed(cb.at[pl.ds(base_v[0], 16)], kv, mask=m)
                d2 = lax.shift_right_logical(kv, s8) & m255
                plsc.addupdate_scatter(hist, [lane_c + d2], ones, mask=m)
                base_v = base_v + plsc.all_reduce_population_count(m)
            return base_v

        ng1 = (n1 + 16 * _UC - 1) // (16 * _UC)
        lax.fori_loop(0, ng1, r2, zeros)
        d2s, n3, k3 = pick(k2)

        def r3(g, c):
            for u in range(_UC):
                i = g * _UC + u
                kv = cb[pl.ds(i * 16, 16)]
                valid = (i * 16 + lane) < n2
                d2 = lax.shift_right_logical(kv, s8) & m255
                m = (d2 == d2s) & valid
                d3 = kv & m255
                plsc.addupdate_scatter(hist, [lane_c + d3], ones, mask=m)
            return c

        ng2 = (n2 + 16 * _UC - 1) // (16 * _UC)
        lax.fori_loop(0, ng2, r3, 0)
        d3s, _, _ = pick(k3)

        tkey = (d0s * 16777216) + (d1s * 65536) + (d2s * 256) + d3s
        tbits = jnp.where(tkey < 0, tkey ^ _MIN32, ~tkey)
        tfv = lax.bitcast_convert_type(jnp.broadcast_to(tbits, (16,)), jnp.float32)

        def w(g, c):
            for u in range(_U):
                i = g * _U + u
                v = xv[pl.ds(i * 16, 16)]
                xv[pl.ds(i * 16, 16)] = jnp.where(v < tfv, neg_inf, v)
            return c

        lax.fori_loop(0, _NV // _U, w, 0)
        pltpu.sync_copy(xv, out_hbm.at[row])
        return carry

    zero_hist()
    lax.fori_loop(0, _RPW, do_row, 0)


@jax.jit
def kernel(x):
    mesh = plsc.VectorSubcoreMesh(
        core_axis_name="c", subcore_axis_name="s", num_cores=2, num_subcores=16
    )
    f = pl.kernel(
        _sc_body,
        out_type=jax.ShapeDtypeStruct((_B, _N), jnp.float32),
        mesh=mesh,
        compiler_params=pltpu.CompilerParams(needs_layout_passes=False),
        scratch_types=[
            pltpu.VMEM((_N,), jnp.float32),
            pltpu.VMEM((_N + 16,), jnp.int32),
            pltpu.VMEM((4096,), jnp.int32),
            pltpu.VMEM((272,), jnp.int32),
            pltpu.VMEM((272,), jnp.int32),
        ],
    )
    return f(x)


# speedup vs baseline: 1.3301x; 1.1491x over previous
import functools
import jax
import jax.numpy as jnp
from jax import lax
from jax.experimental import pallas as pl
from jax.experimental.pallas import tpu as pltpu
from jax.experimental.pallas import tpu_sc as plsc

_K = 50
_B = 128
_N = 32768
_NV = _N // 16
_NW = 32
_RPW = _B // _NW
_MIN32 = -2147483648
_U = 8
_UC = 4


def _sc_body(x_hbm, out_hbm, xv, cb, hist, tot, ss):
    nc = 2
    wid = lax.axis_index("s") * nc + lax.axis_index("c")
    lane = lax.iota(jnp.int32, 16)
    lane_c = lane * 257
    ones = jnp.ones((16,), jnp.int32)
    zeros = jnp.zeros((16,), jnp.int32)
    s24 = jnp.full((16,), 24, jnp.int32)
    s16 = jnp.full((16,), 16, jnp.int32)
    s8 = jnp.full((16,), 8, jnp.int32)
    m255 = jnp.full((16,), 255, jnp.int32)
    neg_inf = jnp.full((16,), -jnp.inf, jnp.float32)

    def key_of(v):
        bi = lax.bitcast_convert_type(v, jnp.int32)
        return bi ^ ((bi >> 31) | _MIN32)

    def zero_hist():
        def zh(i, c):
            hist[pl.ds(i * 16, 16)] = zeros
            return c
        lax.fori_loop(0, 258, zh, 0)

    def pick(k_rem):
        def tj(j, c):
            base = j * 16
            acc = hist[pl.ds(base, 16)]
            hist[pl.ds(base, 16)] = zeros
            for l in range(1, 16):
                off = l * 257 + base
                acc = acc + hist[pl.ds(off, 16)]
                hist[pl.ds(off, 16)] = zeros
            tot[pl.ds(base, 16)] = acc
            return c
        lax.fori_loop(0, 16, tj, 0)

        def sj(i, carry):
            c_above, cnt_v = carry
            j = 15 - i
            v = tot[pl.ds(j * 16, 16)]
            cum = plsc.cumsum(v)
            s = cum[15]
            ssv = (s + c_above) - cum + v
            ss[pl.ds(j * 16, 16)] = ssv
            cnt_v = cnt_v + plsc.all_reduce_population_count(ssv >= k_rem)
            return (c_above + s, cnt_v)

        _, cnt_v = lax.fori_loop(0, 16, sj, (jnp.int32(0), zeros))
        dstar = cnt_v[0] - 1
        t_d = tot[pl.ds(dstar, 16)][0]
        ss_d = ss[pl.ds(dstar, 16)][0]
        k_next = k_rem - (ss_d - t_d)
        return dstar, t_d, k_next

    def do_row(r, carry):
        row = wid * _RPW + r
        pltpu.sync_copy(x_hbm.at[row], xv)

        def h0(g, c):
            for u in range(_U):
                v = xv[pl.ds((g * _U + u) * 16, 16)]
                d0 = lax.shift_right_logical(key_of(v), s24)
                plsc.addupdate_scatter(hist, [lane_c + d0], ones)
            return c

        lax.fori_loop(0, _NV // _U, h0, 0)
        d0s, n1, k1 = pick(jnp.int32(_K))

        def r1(g, base_v):
            for u in range(_U):
                v = xv[pl.ds((g * _U + u) * 16, 16)]
                key = key_of(v)
                d0 = lax.shift_right_logical(key, s24)
                m = d0 == d0s
                plsc.store_compressed(cb.at[pl.ds(base_v[0], 16)], key, mask=m)
                d1 = lax.shift_right_logical(key, s16) & m255
                plsc.addupdate_scatter(hist, [lane_c + d1], ones, mask=m)
                base_v = base_v + plsc.all_reduce_population_count(m)
            return base_v

        lax.fori_loop(0, _NV // _U, r1, zeros)
        d1s, n2, k2 = pick(k1)

        def r2(g, base_v):
            for u in range(_UC):
                i = g * _UC + u
                kv = cb[pl.ds(i * 16, 16)]
                valid = (i * 16 + lane) < n1
                d1 = lax.shift_right_logical(kv, s16) & m255
                m = (d1 == d1s) & valid
                plsc.store_compressed(cb.at[pl.ds(base_v[0], 16)], kv, mask=m)
                d2 = lax.shift_right_logical(kv, s8) & m255
                plsc.addupdate_scatter(hist, [lane_c + d2], ones, mask=m)
                base_v = base_v + plsc.all_reduce_population_count(m)
            return base_v

        ng1 = (n1 + 16 * _UC - 1) // (16 * _UC)
        lax.fori_loop(0, ng1, r2, zeros)
        d2s, n3, k3 = pick(k2)

        def r3(g, c):
            for u in range(_UC):
                i = g * _UC + u
                kv = cb[pl.ds(i * 16, 16)]
                valid = (i * 16 + lane) < n2
                d2 = lax.shift_right_logical(kv, s8) & m255
                m = (d2 == d2s) & valid
                d3 = kv & m255
                plsc.addupdate_scatter(hist, [lane_c + d3], ones, mask=m)
            return c

        ng2 = (n2 + 16 * _UC - 1) // (16 * _UC)
        lax.fori_loop(0, ng2, r3, 0)
        d3s, _, _ = pick(k3)

        tkey = (d0s * 16777216) + (d1s * 65536) + (d2s * 256) + d3s
        tbits = jnp.where(tkey < 0, tkey ^ _MIN32, ~tkey)
        tfv = lax.bitcast_convert_type(jnp.broadcast_to(tbits, (16,)), jnp.float32)

        def w(g, c):
            for u in range(_U):
                i = g * _U + u
                v = xv[pl.ds(i * 16, 16)]
                xv[pl.ds(i * 16, 16)] = jnp.where(v < tfv, neg_inf, v)
            return c

        lax.fori_loop(0, _NV // _U, w, 0)
        pltpu.sync_copy(xv, out_hbm.at[row])
        return carry

    zero_hist()
    lax.fori_loop(0, _RPW, do_row, 0)


@jax.jit
def kernel(x):
    mesh = plsc.VectorSubcoreMesh(
        core_axis_name="c", subcore_axis_name="s", num_cores=2, num_subcores=16
    )
    f = pl.kernel(
        _sc_body,
        out_type=jax.ShapeDtypeStruct((_B, _N), jnp.float32),
        mesh=mesh,
        compiler_params=pltpu.CompilerParams(needs_layout_passes=False),
        scratch_types=[
            pltpu.VMEM((_N,), jnp.float32),
            pltpu.VMEM((_N + 16,), jnp.int32),
            pltpu.VMEM((4128,), jnp.int32),
            pltpu.VMEM((272,), jnp.int32),
            pltpu.VMEM((272,), jnp.int32),
        ],
    )
    return f(x)


# speedup vs baseline: 3.5711x; 2.6848x over previous
import functools
import jax
import jax.numpy as jnp
from jax import lax
from jax.experimental import pallas as pl
from jax.experimental.pallas import tpu as pltpu
from jax.experimental.pallas import tpu_sc as plsc

_K = 50
_B = 128
_N = 32768
_NV = _N // 16
_NW = 32
_RPW = _B // _NW
_MIN32 = -2147483648
_U = 8
_UC = 4


def _sc_body(x_hbm, out_hbm, xv, cb, hist, tot, ss):
    nc = 2
    wid = lax.axis_index("s") * nc + lax.axis_index("c")
    lane = lax.iota(jnp.int32, 16)
    lane_c = lane * 257
    ones = jnp.ones((16,), jnp.int32)
    zeros = jnp.zeros((16,), jnp.int32)
    s24 = jnp.full((16,), 24, jnp.int32)
    s16 = jnp.full((16,), 16, jnp.int32)
    s8 = jnp.full((16,), 8, jnp.int32)
    m255 = jnp.full((16,), 255, jnp.int32)
    neg_inf = jnp.full((16,), -jnp.inf, jnp.float32)

    def key_of(v):
        bi = lax.bitcast_convert_type(v, jnp.int32)
        return bi ^ ((bi >> 31) | _MIN32)

    def zero_hist():
        def zh(i, c):
            hist[pl.ds(i * 16, 16)] = zeros
            return c
        lax.fori_loop(0, 258, zh, 0)

    def pick(k_rem):
        @plsc.parallel_loop(0, 16)
        def tj(j):
            base = j * 16
            acc = hist[pl.ds(base, 16)]
            hist[pl.ds(base, 16)] = zeros
            for l in range(1, 16):
                off = l * 257 + base
                acc = acc + hist[pl.ds(off, 16)]
                hist[pl.ds(off, 16)] = zeros
            tot[pl.ds(base, 16)] = acc

        def sj(i, carry):
            c_above, cnt_v = carry
            j = 15 - i
            v = tot[pl.ds(j * 16, 16)]
            cum = plsc.cumsum(v)
            s = cum[15]
            ssv = (s + c_above) - cum + v
            ss[pl.ds(j * 16, 16)] = ssv
            cnt_v = cnt_v + plsc.all_reduce_population_count(ssv >= k_rem)
            return (c_above + s, cnt_v)

        _, cnt_v = lax.fori_loop(0, 16, sj, (jnp.int32(0), zeros))
        dstar = cnt_v[0] - 1
        t_d = tot[pl.ds(dstar, 16)][0]
        ss_d = ss[pl.ds(dstar, 16)][0]
        k_next = k_rem - (ss_d - t_d)
        return dstar, t_d, k_next

    def do_row(r, carry):
        row = wid * _RPW + r
        pltpu.sync_copy(x_hbm.at[row], xv)

        @plsc.parallel_loop(0, _NV, unroll=_U)
        def h0(i):
            v = xv[pl.ds(i * 16, 16)]
            d0 = lax.shift_right_logical(key_of(v), s24)
            plsc.addupdate_scatter(hist, [lane_c + d0], ones)

        d0s, n1, k1 = pick(jnp.int32(_K))

        @plsc.parallel_loop(0, _NV, unroll=_U, carry=zeros)
        def r1(i, base_v):
            v = xv[pl.ds(i * 16, 16)]
            key = key_of(v)
            d0 = lax.shift_right_logical(key, s24)
            m = d0 == d0s
            plsc.store_compressed(cb.at[pl.ds(base_v[0], 16)], key, mask=m)
            d1 = lax.shift_right_logical(key, s16) & m255
            plsc.addupdate_scatter(hist, [lane_c + d1], ones, mask=m)
            return base_v + plsc.all_reduce_population_count(m)

        d1s, n2, k2 = pick(k1)

        def r2(g, base_v):
            for u in range(_UC):
                i = g * _UC + u
                kv = cb[pl.ds(i * 16, 16)]
                valid = (i * 16 + lane) < n1
                d1 = lax.shift_right_logical(kv, s16) & m255
                m = (d1 == d1s) & valid
                plsc.store_compressed(cb.at[pl.ds(base_v[0], 16)], kv, mask=m)
                d2 = lax.shift_right_logical(kv, s8) & m255
                plsc.addupdate_scatter(hist, [lane_c + d2], ones, mask=m)
                base_v = base_v + plsc.all_reduce_population_count(m)
            return base_v

        ng1 = (n1 + 16 * _UC - 1) // (16 * _UC)
        lax.fori_loop(0, ng1, r2, zeros)
        d2s, n3, k3 = pick(k2)

        def r3(g, c):
            for u in range(_UC):
                i = g * _UC + u
                kv = cb[pl.ds(i * 16, 16)]
                valid = (i * 16 + lane) < n2
                d2 = lax.shift_right_logical(kv, s8) & m255
                m = (d2 == d2s) & valid
                d3 = kv & m255
                plsc.addupdate_scatter(hist, [lane_c + d3], ones, mask=m)
            return c

        ng2 = (n2 + 16 * _UC - 1) // (16 * _UC)
        lax.fori_loop(0, ng2, r3, 0)
        d3s, _, _ = pick(k3)

        tkey = (d0s * 16777216) + (d1s * 65536) + (d2s * 256) + d3s
        tbits = jnp.where(tkey < 0, tkey ^ _MIN32, ~tkey)
        tfv = lax.bitcast_convert_type(jnp.broadcast_to(tbits, (16,)), jnp.float32)

        @plsc.parallel_loop(0, _NV, unroll=_U)
        def w(i):
            v = xv[pl.ds(i * 16, 16)]
            xv[pl.ds(i * 16, 16)] = jnp.where(v < tfv, neg_inf, v)
        pltpu.sync_copy(xv, out_hbm.at[row])
        return carry

    zero_hist()
    lax.fori_loop(0, _RPW, do_row, 0)


@jax.jit
def kernel(x):
    mesh = plsc.VectorSubcoreMesh(
        core_axis_name="c", subcore_axis_name="s", num_cores=2, num_subcores=16
    )
    f = pl.kernel(
        _sc_body,
        out_type=jax.ShapeDtypeStruct((_B, _N), jnp.float32),
        mesh=mesh,
        compiler_params=pltpu.CompilerParams(needs_layout_passes=False),
        scratch_types=[
            pltpu.VMEM((_N,), jnp.float32),
            pltpu.VMEM((_N + 16,), jnp.int32),
            pltpu.VMEM((4128,), jnp.int32),
            pltpu.VMEM((272,), jnp.int32),
            pltpu.VMEM((272,), jnp.int32),
        ],
    )
    return f(x)


# speedup vs baseline: 3.7818x; 1.0590x over previous
import functools
import jax
import jax.numpy as jnp
from jax import lax
from jax.experimental import pallas as pl
from jax.experimental.pallas import tpu as pltpu
from jax.experimental.pallas import tpu_sc as plsc

_K = 50
_B = 128
_N = 32768
_NV = _N // 16
_NW = 32
_RPW = _B // _NW
_MIN32 = -2147483648
_U = 8
_UC = 4
_NCH = 4


def _sc_body(x_hbm, out_hbm, xv, cb, hist, tot, ss, sem_in, sem_out):
    nc = 2
    wid = lax.axis_index("s") * nc + lax.axis_index("c")
    lane = lax.iota(jnp.int32, 16)
    lane_c = lane * 257
    ones = jnp.ones((16,), jnp.int32)
    zeros = jnp.zeros((16,), jnp.int32)
    s24 = jnp.full((16,), 24, jnp.int32)
    s16 = jnp.full((16,), 16, jnp.int32)
    s8 = jnp.full((16,), 8, jnp.int32)
    m255 = jnp.full((16,), 255, jnp.int32)
    neg_inf = jnp.full((16,), -jnp.inf, jnp.float32)

    def key_of(v):
        bi = lax.bitcast_convert_type(v, jnp.int32)
        return bi ^ ((bi >> 31) | _MIN32)

    def zero_hist():
        def zh(i, c):
            hist[pl.ds(i * 16, 16)] = zeros
            return c
        lax.fori_loop(0, 258, zh, 0)

    def pick(k_rem):
        @plsc.parallel_loop(0, 16)
        def tj(j):
            base = j * 16
            acc = hist[pl.ds(base, 16)]
            hist[pl.ds(base, 16)] = zeros
            for l in range(1, 16):
                off = l * 257 + base
                acc = acc + hist[pl.ds(off, 16)]
                hist[pl.ds(off, 16)] = zeros
            tot[pl.ds(base, 16)] = acc

        def sj(i, carry):
            c_above, cnt_v = carry
            j = 15 - i
            v = tot[pl.ds(j * 16, 16)]
            cum = plsc.cumsum(v)
            s = cum[15]
            ssv = (s + c_above) - cum + v
            ss[pl.ds(j * 16, 16)] = ssv
            cnt_v = cnt_v + plsc.all_reduce_population_count(ssv >= k_rem)
            return (c_above + s, cnt_v)

        _, cnt_v = lax.fori_loop(0, 16, sj, (jnp.int32(0), zeros))
        dstar = cnt_v[0] - 1
        t_d = tot[pl.ds(dstar, 16)][0]
        ss_d = ss[pl.ds(dstar, 16)][0]
        k_next = k_rem - (ss_d - t_d)
        return dstar, t_d, k_next

    def do_row(r, carry):
        row = wid * _RPW + r
        ch = _N // _NCH
        chv = _NV // _NCH

        in_cp = []
        for c in range(_NCH):
            sl = pl.ds(c * ch, ch)

            @pl.when(r > 0)
            def _drain(c=c, sl=sl):
                pltpu.make_async_copy(
                    xv.at[sl], out_hbm.at[row].at[sl], sem_out[c]
                ).wait()

            in_cp.append(
                pltpu.async_copy(x_hbm.at[row].at[sl], xv.at[sl], sem_in[c])
            )

        for c in range(_NCH):
            in_cp[c].wait()

            @plsc.parallel_loop(c * chv, (c + 1) * chv, unroll=_U)
            def h0(i):
                v = xv[pl.ds(i * 16, 16)]
                d0 = lax.shift_right_logical(key_of(v), s24)
                plsc.addupdate_scatter(hist, [lane_c + d0], ones)

        d0s, n1, k1 = pick(jnp.int32(_K))

        @plsc.parallel_loop(0, _NV, unroll=_U, carry=zeros)
        def r1(i, base_v):
            v = xv[pl.ds(i * 16, 16)]
            key = key_of(v)
            d0 = lax.shift_right_logical(key, s24)
            m = d0 == d0s
            plsc.store_compressed(cb.at[pl.ds(base_v[0], 16)], key, mask=m)
            d1 = lax.shift_right_logical(key, s16) & m255
            plsc.addupdate_scatter(hist, [lane_c + d1], ones, mask=m)
            return base_v + plsc.all_reduce_population_count(m)

        d1s, n2, k2 = pick(k1)

        def r2(g, base_v):
            for u in range(_UC):
                i = g * _UC + u
                kv = cb[pl.ds(i * 16, 16)]
                valid = (i * 16 + lane) < n1
                d1 = lax.shift_right_logical(kv, s16) & m255
                m = (d1 == d1s) & valid
                plsc.store_compressed(cb.at[pl.ds(base_v[0], 16)], kv, mask=m)
                d2 = lax.shift_right_logical(kv, s8) & m255
                plsc.addupdate_scatter(hist, [lane_c + d2], ones, mask=m)
                base_v = base_v + plsc.all_reduce_population_count(m)
            return base_v

        ng1 = (n1 + 16 * _UC - 1) // (16 * _UC)
        lax.fori_loop(0, ng1, r2, zeros)
        d2s, n3, k3 = pick(k2)

        def r3(g, c):
            for u in range(_UC):
                i = g * _UC + u
                kv = cb[pl.ds(i * 16, 16)]
                valid = (i * 16 + lane) < n2
                d2 = lax.shift_right_logical(kv, s8) & m255
                m = (d2 == d2s) & valid
                d3 = kv & m255
                plsc.addupdate_scatter(hist, [lane_c + d3], ones, mask=m)
            return c

        ng2 = (n2 + 16 * _UC - 1) // (16 * _UC)
        lax.fori_loop(0, ng2, r3, 0)
        d3s, _, _ = pick(k3)

        tkey = (d0s * 16777216) + (d1s * 65536) + (d2s * 256) + d3s
        tbits = jnp.where(tkey < 0, tkey ^ _MIN32, ~tkey)
        tfv = lax.bitcast_convert_type(jnp.broadcast_to(tbits, (16,)), jnp.float32)

        for c in range(_NCH):

            @plsc.parallel_loop(c * chv, (c + 1) * chv, unroll=_U)
            def w(i):
                v = xv[pl.ds(i * 16, 16)]
                xv[pl.ds(i * 16, 16)] = jnp.where(v < tfv, neg_inf, v)

            sl = pl.ds(c * ch, ch)
            pltpu.async_copy(xv.at[sl], out_hbm.at[row].at[sl], sem_out[c])
        return carry

    zero_hist()
    lax.fori_loop(0, _RPW, do_row, 0)
    last = wid * _RPW + (_RPW - 1)
    for c in range(_NCH):
        sl = pl.ds(c * (_N // _NCH), _N // _NCH)
        pltpu.make_async_copy(xv.at[sl], out_hbm.at[last].at[sl], sem_out[c]).wait()


@jax.jit
def kernel(x):
    mesh = plsc.VectorSubcoreMesh(
        core_axis_name="c", subcore_axis_name="s", num_cores=2, num_subcores=16
    )
    f = pl.kernel(
        _sc_body,
        out_type=jax.ShapeDtypeStruct((_B, _N), jnp.float32),
        mesh=mesh,
        compiler_params=pltpu.CompilerParams(needs_layout_passes=False),
        scratch_types=[
            pltpu.VMEM((_N,), jnp.float32),
            pltpu.VMEM((_N + 16,), jnp.int32),
            pltpu.VMEM((4128,), jnp.int32),
            pltpu.VMEM((272,), jnp.int32),
            pltpu.VMEM((272,), jnp.int32),
            [pltpu.SemaphoreType.DMA] * _NCH,
            [pltpu.SemaphoreType.DMA] * _NCH,
        ],
    )
    return f(x)


# speedup vs baseline: 3.9146x; 1.0351x over previous
import functools
import jax
import jax.numpy as jnp
from jax import lax
from jax.experimental import pallas as pl
from jax.experimental.pallas import tpu as pltpu
from jax.experimental.pallas import tpu_sc as plsc

_K = 50
_B = 128
_N = 32768
_NV = _N // 16
_NW = 32
_RPW = _B // _NW
_MIN32 = -2147483648
_U = 8
_UC = 4
_NCH = 4


def _sc_body(x_hbm, out_hbm, xv, cb, hist, tot, ss, sem_in, sem_out):
    nc = 2
    wid = lax.axis_index("s") * nc + lax.axis_index("c")
    lane = lax.iota(jnp.int32, 16)
    lane_c = lane * 257
    ones = jnp.ones((16,), jnp.int32)
    zeros = jnp.zeros((16,), jnp.int32)
    s24 = jnp.full((16,), 24, jnp.int32)
    s16 = jnp.full((16,), 16, jnp.int32)
    s8 = jnp.full((16,), 8, jnp.int32)
    m255 = jnp.full((16,), 255, jnp.int32)
    neg_inf = jnp.full((16,), -jnp.inf, jnp.float32)

    def key_of(v):
        bi = lax.bitcast_convert_type(v, jnp.int32)
        return bi ^ ((bi >> 31) | _MIN32)

    def zero_hist():
        def zh(i, c):
            hist[pl.ds(i * 16, 16)] = zeros
            return c
        lax.fori_loop(0, 258, zh, 0)

    def pick(k_rem):
        @plsc.parallel_loop(0, 16)
        def tj(j):
            base = j * 16
            acc = hist[pl.ds(base, 16)]
            hist[pl.ds(base, 16)] = zeros
            for l in range(1, 16):
                off = l * 257 + base
                acc = acc + hist[pl.ds(off, 16)]
                hist[pl.ds(off, 16)] = zeros
            tot[pl.ds(base, 16)] = acc

        def sj(i, carry):
            c_above, cnt_v = carry
            j = 15 - i
            v = tot[pl.ds(j * 16, 16)]
            cum = plsc.cumsum(v)
            s = cum[15]
            ssv = (s + c_above) - cum + v
            ss[pl.ds(j * 16, 16)] = ssv
            cnt_v = cnt_v + plsc.all_reduce_population_count(ssv >= k_rem)
            return (c_above + s, cnt_v)

        _, cnt_v = lax.fori_loop(0, 16, sj, (jnp.int32(0), zeros))
        dstar = cnt_v[0] - 1
        t_d = tot[pl.ds(dstar, 16)][0]
        ss_d = ss[pl.ds(dstar, 16)][0]
        k_next = k_rem - (ss_d - t_d)
        return dstar, t_d, k_next

    def do_row(r, carry):
        row = wid * _RPW + r
        ch = _N // _NCH
        chv = _NV // _NCH

        in_cp = []
        for c in range(_NCH):
            sl = pl.ds(c * ch, ch)

            @pl.when(r > 0)
            def _drain(c=c, sl=sl):
                pltpu.make_async_copy(
                    xv.at[sl], out_hbm.at[row].at[sl], sem_out[c]
                ).wait()

            in_cp.append(
                pltpu.async_copy(x_hbm.at[row].at[sl], xv.at[sl], sem_in[c])
            )

        for c in range(_NCH):
            in_cp[c].wait()

            @plsc.parallel_loop(c * chv, (c + 1) * chv, unroll=_U)
            def h0(i):
                v = xv[pl.ds(i * 16, 16)]
                d0 = lax.shift_right_logical(key_of(v), s24)
                plsc.addupdate_scatter(hist, [lane_c + d0], ones)

        d0s, n1, k1 = pick(jnp.int32(_K))

        @plsc.parallel_loop(0, _NV, unroll=_U, carry=zeros)
        def r1(i, base_v):
            v = xv[pl.ds(i * 16, 16)]
            key = key_of(v)
            d0 = lax.shift_right_logical(key, s24)
            m = d0 == d0s
            plsc.store_compressed(cb.at[pl.ds(base_v[0], 16)], key, mask=m)
            return base_v + plsc.all_reduce_population_count(m)

        def h1(g, c):
            for u in range(_UC):
                i = g * _UC + u
                kv = cb[pl.ds(i * 16, 16)]
                valid = (i * 16 + lane) < n1
                d1 = lax.shift_right_logical(kv, s16) & m255
                plsc.addupdate_scatter(hist, [lane_c + d1], ones, mask=valid)
            return c

        ng0 = (n1 + 16 * _UC - 1) // (16 * _UC)
        lax.fori_loop(0, ng0, h1, 0)
        d1s, n2, k2 = pick(k1)

        def r2(g, base_v):
            for u in range(_UC):
                i = g * _UC + u
                kv = cb[pl.ds(i * 16, 16)]
                valid = (i * 16 + lane) < n1
                d1 = lax.shift_right_logical(kv, s16) & m255
                m = (d1 == d1s) & valid
                plsc.store_compressed(cb.at[pl.ds(base_v[0], 16)], kv, mask=m)
                d2 = lax.shift_right_logical(kv, s8) & m255
                plsc.addupdate_scatter(hist, [lane_c + d2], ones, mask=m)
                base_v = base_v + plsc.all_reduce_population_count(m)
            return base_v

        ng1 = (n1 + 16 * _UC - 1) // (16 * _UC)
        lax.fori_loop(0, ng1, r2, zeros)
        d2s, n3, k3 = pick(k2)

        def r3(g, c):
            for u in range(_UC):
                i = g * _UC + u
                kv = cb[pl.ds(i * 16, 16)]
                valid = (i * 16 + lane) < n2
                d2 = lax.shift_right_logical(kv, s8) & m255
                m = (d2 == d2s) & valid
                d3 = kv & m255
                plsc.addupdate_scatter(hist, [lane_c + d3], ones, mask=m)
            return c

        ng2 = (n2 + 16 * _UC - 1) // (16 * _UC)
        lax.fori_loop(0, ng2, r3, 0)
        d3s, _, _ = pick(k3)

        tkey = (d0s * 16777216) + (d1s * 65536) + (d2s * 256) + d3s
        tbits = jnp.where(tkey < 0, tkey ^ _MIN32, ~tkey)
        tfv = lax.bitcast_convert_type(jnp.broadcast_to(tbits, (16,)), jnp.float32)

        for c in range(_NCH):

            @plsc.parallel_loop(c * chv, (c + 1) * chv, unroll=_U)
            def w(i):
                v = xv[pl.ds(i * 16, 16)]
                xv[pl.ds(i * 16, 16)] = jnp.where(v < tfv, neg_inf, v)

            sl = pl.ds(c * ch, ch)
            pltpu.async_copy(xv.at[sl], out_hbm.at[row].at[sl], sem_out[c])
        return carry

    zero_hist()
    lax.fori_loop(0, _RPW, do_row, 0)
    last = wid * _RPW + (_RPW - 1)
    for c in range(_NCH):
        sl = pl.ds(c * (_N // _NCH), _N // _NCH)
        pltpu.make_async_copy(xv.at[sl], out_hbm.at[last].at[sl], sem_out[c]).wait()


@jax.jit
def kernel(x):
    mesh = plsc.VectorSubcoreMesh(
        core_axis_name="c", subcore_axis_name="s", num_cores=2, num_subcores=16
    )
    f = pl.kernel(
        _sc_body,
        out_type=jax.ShapeDtypeStruct((_B, _N), jnp.float32),
        mesh=mesh,
        compiler_params=pltpu.CompilerParams(needs_layout_passes=False),
        scratch_types=[
            pltpu.VMEM((_N,), jnp.float32),
            pltpu.VMEM((_N + 16,), jnp.int32),
            pltpu.VMEM((4128,), jnp.int32),
            pltpu.VMEM((272,), jnp.int32),
            pltpu.VMEM((272,), jnp.int32),
            [pltpu.SemaphoreType.DMA] * _NCH,
            [pltpu.SemaphoreType.DMA] * _NCH,
        ],
    )
    return f(x)


# speedup vs baseline: 4.0213x; 1.0272x over previous
import functools
import jax
import jax.numpy as jnp
from jax import lax
from jax.experimental import pallas as pl
from jax.experimental.pallas import tpu as pltpu
from jax.experimental.pallas import tpu_sc as plsc

_K = 50
_B = 128
_N = 32768
_NV = _N // 16
_NW = 32
_RPW = _B // _NW
_MIN32 = -2147483648
_U = 8
_UC = 4
_NCH = 4


def _sc_body(x_hbm, out_hbm, xv, cb, hist, tot, ss, sem_in, sem_out):
    nc = 2
    wid = lax.axis_index("s") * nc + lax.axis_index("c")
    lane = lax.iota(jnp.int32, 16)
    lane_c = lane * 257
    ones = jnp.ones((16,), jnp.int32)
    zeros = jnp.zeros((16,), jnp.int32)
    s24 = jnp.full((16,), 24, jnp.int32)
    s16 = jnp.full((16,), 16, jnp.int32)
    s8 = jnp.full((16,), 8, jnp.int32)
    m255 = jnp.full((16,), 255, jnp.int32)
    neg_inf = jnp.full((16,), -jnp.inf, jnp.float32)

    zf = jnp.zeros((16,), jnp.float32)

    def key_of(v):
        bi = lax.bitcast_convert_type(v + zf, jnp.int32)
        return bi ^ ((bi >> 31) | _MIN32)

    def digit_bound(key_s):
        fbits = jnp.where(key_s < 0, key_s ^ _MIN32, ~key_s)
        return lax.bitcast_convert_type(
            jnp.broadcast_to(fbits, (16,)), jnp.float32
        )

    def zero_hist():
        def zh(i, c):
            hist[pl.ds(i * 16, 16)] = zeros
            return c
        lax.fori_loop(0, 258, zh, 0)

    def pick(k_rem):
        @plsc.parallel_loop(0, 16)
        def tj(j):
            base = j * 16
            acc = hist[pl.ds(base, 16)]
            hist[pl.ds(base, 16)] = zeros
            for l in range(1, 16):
                off = l * 257 + base
                acc = acc + hist[pl.ds(off, 16)]
                hist[pl.ds(off, 16)] = zeros
            tot[pl.ds(base, 16)] = acc

        def sj(i, carry):
            c_above, cnt_v = carry
            j = 15 - i
            v = tot[pl.ds(j * 16, 16)]
            cum = plsc.cumsum(v)
            s = cum[15]
            ssv = (s + c_above) - cum + v
            ss[pl.ds(j * 16, 16)] = ssv
            cnt_v = cnt_v + plsc.all_reduce_population_count(ssv >= k_rem)
            return (c_above + s, cnt_v)

        _, cnt_v = lax.fori_loop(0, 16, sj, (jnp.int32(0), zeros))
        dstar = cnt_v[0] - 1
        t_d = tot[pl.ds(dstar, 16)][0]
        ss_d = ss[pl.ds(dstar, 16)][0]
        k_next = k_rem - (ss_d - t_d)
        return dstar, t_d, k_next

    def do_row(r, carry):
        row = wid * _RPW + r
        ch = _N // _NCH
        chv = _NV // _NCH

        in_cp = []
        for c in range(_NCH):
            sl = pl.ds(c * ch, ch)

            @pl.when(r > 0)
            def _drain(c=c, sl=sl):
                pltpu.make_async_copy(
                    xv.at[sl], out_hbm.at[row].at[sl], sem_out[c]
                ).wait()

            in_cp.append(
                pltpu.async_copy(x_hbm.at[row].at[sl], xv.at[sl], sem_in[c])
            )

        for c in range(_NCH):
            in_cp[c].wait()

            @plsc.parallel_loop(c * chv, (c + 1) * chv, unroll=_U)
            def h0(i):
                v = xv[pl.ds(i * 16, 16)]
                d0 = lax.shift_right_logical(key_of(v), s24)
                plsc.addupdate_scatter(hist, [lane_c + d0], ones)

        d0s, n1, k1 = pick(jnp.int32(_K))

        lo_v = digit_bound(d0s * 16777216)
        hi_v = jnp.where(
            jnp.broadcast_to(d0s, (16,)) == 255,
            lax.bitcast_convert_type(
                jnp.full((16,), 0x7F800000, jnp.int32), jnp.float32
            ),
            digit_bound((d0s + 1) * 16777216),
        )

        @plsc.parallel_loop(0, _NV, unroll=_U, carry=zeros)
        def r1(i, base_v):
            v = xv[pl.ds(i * 16, 16)]
            m = (v >= lo_v) & (v < hi_v)
            plsc.store_compressed(cb.at[pl.ds(base_v[0], 16)], v, mask=m)
            return base_v + plsc.all_reduce_population_count(m)

        def h1(g, c):
            for u in range(_UC):
                i = g * _UC + u
                kv = key_of(cb[pl.ds(i * 16, 16)])
                valid = (i * 16 + lane) < n1
                d1 = lax.shift_right_logical(kv, s16) & m255
                plsc.addupdate_scatter(hist, [lane_c + d1], ones, mask=valid)
            return c

        ng0 = (n1 + 16 * _UC - 1) // (16 * _UC)
        lax.fori_loop(0, ng0, h1, 0)
        d1s, n2, k2 = pick(k1)

        def r2(g, base_v):
            for u in range(_UC):
                i = g * _UC + u
                vv = cb[pl.ds(i * 16, 16)]
                kv = key_of(vv)
                valid = (i * 16 + lane) < n1
                d1 = lax.shift_right_logical(kv, s16) & m255
                m = (d1 == d1s) & valid
                plsc.store_compressed(cb.at[pl.ds(base_v[0], 16)], vv, mask=m)
                d2 = lax.shift_right_logical(kv, s8) & m255
                plsc.addupdate_scatter(hist, [lane_c + d2], ones, mask=m)
                base_v = base_v + plsc.all_reduce_population_count(m)
            return base_v

        ng1 = (n1 + 16 * _UC - 1) // (16 * _UC)
        lax.fori_loop(0, ng1, r2, zeros)
        d2s, n3, k3 = pick(k2)

        def r3(g, c):
            for u in range(_UC):
                i = g * _UC + u
                kv = key_of(cb[pl.ds(i * 16, 16)])
                valid = (i * 16 + lane) < n2
                d2 = lax.shift_right_logical(kv, s8) & m255
                m = (d2 == d2s) & valid
                d3 = kv & m255
                plsc.addupdate_scatter(hist, [lane_c + d3], ones, mask=m)
            return c

        ng2 = (n2 + 16 * _UC - 1) // (16 * _UC)
        lax.fori_loop(0, ng2, r3, 0)
        d3s, _, _ = pick(k3)

        tkey = (d0s * 16777216) + (d1s * 65536) + (d2s * 256) + d3s
        tbits = jnp.where(tkey < 0, tkey ^ _MIN32, ~tkey)
        tfv = lax.bitcast_convert_type(jnp.broadcast_to(tbits, (16,)), jnp.float32)

        for c in range(_NCH):

            @plsc.parallel_loop(c * chv, (c + 1) * chv, unroll=_U)
            def w(i):
                v = xv[pl.ds(i * 16, 16)]
                xv[pl.ds(i * 16, 16)] = jnp.where(v < tfv, neg_inf, v)

            sl = pl.ds(c * ch, ch)
            pltpu.async_copy(xv.at[sl], out_hbm.at[row].at[sl], sem_out[c])
        return carry

    zero_hist()
    lax.fori_loop(0, _RPW, do_row, 0)
    last = wid * _RPW + (_RPW - 1)
    for c in range(_NCH):
        sl = pl.ds(c * (_N // _NCH), _N // _NCH)
        pltpu.make_async_copy(xv.at[sl], out_hbm.at[last].at[sl], sem_out[c]).wait()


@jax.jit
def kernel(x):
    mesh = plsc.VectorSubcoreMesh(
        core_axis_name="c", subcore_axis_name="s", num_cores=2, num_subcores=16
    )
    f = pl.kernel(
        _sc_body,
        out_type=jax.ShapeDtypeStruct((_B, _N), jnp.float32),
        mesh=mesh,
        compiler_params=pltpu.CompilerParams(needs_layout_passes=False),
        scratch_types=[
            pltpu.VMEM((_N,), jnp.float32),
            pltpu.VMEM((_N + 16,), jnp.float32),
            pltpu.VMEM((4128,), jnp.int32),
            pltpu.VMEM((272,), jnp.int32),
            pltpu.VMEM((272,), jnp.int32),
            [pltpu.SemaphoreType.DMA] * _NCH,
            [pltpu.SemaphoreType.DMA] * _NCH,
        ],
    )
    return f(x)
